# baseline (speedup 1.0000x reference)
import functools

import jax
import jax.numpy as jnp
from jax import lax
from jax.experimental import pallas as pl
from jax.experimental.pallas import tpu as pltpu
from jax.experimental.pallas import tpu_sc as plsc

_K = 1024
_ITERS = 10
_N = 16384
_D = 256
_NS = 16
_DS = _D // _NS
_PPT = _N // 2
_JROW = 128
_JIN = _PPT // _JROW
_AW = _K * _DS

_ROWS = 1024
_NBLK = _N // _ROWS


_sc_mesh = plsc.VectorSubcoreMesh(core_axis_name="c", subcore_axis_name="s")


@functools.partial(
    pl.kernel,
    out_type=jax.ShapeDtypeStruct((_NS, 2, _AW), jnp.float32),
    mesh=_sc_mesh,
    compiler_params=pltpu.CompilerParams(needs_layout_passes=False),
    scratch_types=[
        pltpu.VMEM((_JROW * _DS,), jnp.float32),
        pltpu.VMEM((_JIN, _JROW), jnp.int32),
        pltpu.VMEM((_AW,), jnp.float32),
    ],
)
def _sc_segsum(xs, lab2d, zc, out, data_v, idx_v, acc):
    c = lax.axis_index("c")
    s = lax.axis_index("s")
    w = c * 16 + s
    sl = w // 2
    p = w % 2
    base = p * _PPT * _DS
    pltpu.sync_copy(zc, acc)
    pltpu.sync_copy(lab2d.at[pl.ds(p * _JIN, _JIN)], idx_v)
    cols0 = lax.iota(jnp.int32, 16)

    def chunk(j, carry):
        pltpu.sync_copy(
            xs.at[sl, pl.ds(base + j * (_JROW * _DS), _JROW * _DS)], data_v)

        def grp(g, carry2):
            lvec = idx_v[j, pl.ds(g * 16, 16)] * _DS
            for t in range(16):
                idx = lax.broadcast(lvec[t], (16,)) + cols0
                vec = data_v[pl.ds((g * 16 + t) * _DS, 16)]
                plsc.addupdate_scatter(acc, [idx], vec)
            return carry2

        return lax.fori_loop(0, _JROW // 16, grp, carry)

    lax.fori_loop(0, _JIN, chunk, 0)
    pltpu.sync_copy(acc, out.at[sl, p])


def _assign_body(craw_ref, x_ref, out_ref, cn_ref):
    @pl.when(pl.program_id(0) == 0)
    def _():
        slices = [craw_ref[q, 0] + craw_ref[q, 1] for q in range(_NS)]
        c = jnp.concatenate(slices, axis=1)
        norm = jnp.sqrt(jnp.sum(c * c, axis=1, keepdims=True))
        cn_ref[...] = c / (norm + 1e-12)

    sims = jax.lax.dot_general(
        x_ref[...], cn_ref[...],
        dimension_numbers=(((1,), (1,)), ((), ())),
        preferred_element_type=jnp.float32)
    out_ref[...] = jnp.argmax(sims, axis=1).astype(jnp.int32).reshape(1, 1, _ROWS)


def _tc_assign(xn, craw):
    out = pl.pallas_call(
        _assign_body,
        grid=(_NBLK,),
        in_specs=[
            pl.BlockSpec((_NS, 2, _K, _DS), lambda i: (0, 0, 0, 0)),
            pl.BlockSpec((_ROWS, _D), lambda i: (i, 0)),
        ],
        out_specs=pl.BlockSpec((1, 1, _ROWS), lambda i: (i, 0, 0)),
        out_shape=jax.ShapeDtypeStruct((_NBLK, 1, _ROWS), jnp.int32),
        scratch_shapes=[pltpu.VMEM((_K, _D), jnp.float32)],
    )(craw, xn)
    return out.reshape(_N)


def _norm(v, axis=-1, eps=1e-12):
    return v / (jnp.linalg.norm(v, axis=axis, keepdims=True) + eps)


def kernel(embeddings, batch_indices):
    key = jax.random.key(42)
    embs = embeddings.reshape(-1, embeddings.shape[-1])
    n = embs.shape[0]
    key, sub = jax.random.split(key)
    init_labels = jax.random.randint(sub, (n,), 0, _K)
    x = _norm(embs)
    xs = x.reshape(_N, _NS, _DS).transpose(1, 0, 2).reshape(_NS, _N * _DS)
    zc = jnp.zeros((_AW,), jnp.float32)
    labels = init_labels
    for _ in range(_ITERS):
        lab2d = labels.reshape(_N // _JROW, _JROW)
        craw = _sc_segsum(xs, lab2d, zc)
        labels = _tc_assign(x, craw.reshape(_NS, 2, _K, _DS))
    return (labels, batch_indices.astype(labels.dtype))


# speedup vs baseline: 1.3794x; 1.3794x over previous
import functools

import jax
import jax.numpy as jnp
from jax import lax
from jax.experimental import pallas as pl
from jax.experimental.pallas import tpu as pltpu
from jax.experimental.pallas import tpu_sc as plsc

_K = 1024
_ITERS = 10
_N = 16384
_D = 256
_NS = 16
_DS = _D // _NS
_PPT = _N // 2
_CH = 2048
_NCH = _PPT // _CH
_CW = _CH * _DS
_AW = _K * _DS

_ROWS = 1024
_NBLK = _N // _ROWS


_sc_mesh = plsc.VectorSubcoreMesh(core_axis_name="c", subcore_axis_name="s")


@functools.partial(
    pl.kernel,
    out_type=jax.ShapeDtypeStruct((_NS, 2, _AW), jnp.float32),
    mesh=_sc_mesh,
    compiler_params=pltpu.CompilerParams(needs_layout_passes=False),
    scratch_types=[
        pltpu.VMEM((_CW,), jnp.float32),
        pltpu.VMEM((_CW,), jnp.float32),
        pltpu.VMEM((_PPT // 128, 128), jnp.int32),
        pltpu.VMEM((_AW,), jnp.float32),
        pltpu.SemaphoreType.DMA,
        pltpu.SemaphoreType.DMA,
    ],
)
def _sc_segsum(xs, lab2d, zc, out, data_a, data_b, idx_v, acc, sem0, sem1):
    c = lax.axis_index("c")
    s = lax.axis_index("s")
    w = c * 16 + s
    sl = w // 2
    p = w % 2
    base = p * _PPT * _DS
    sems = (sem0, sem1)
    bufs = (data_a, data_b)
    copies = [
        pltpu.async_copy(
            xs.at[sl, pl.ds(base + j * _CW, _CW)], bufs[j % 2], sems[j % 2])
        for j in range(1)
    ]
    pltpu.sync_copy(zc, acc)
    pltpu.sync_copy(lab2d.at[pl.ds(p * (_PPT // 128), _PPT // 128)], idx_v)
    cols0 = lax.iota(jnp.int32, 16)

    for j in range(_NCH):
        if j + 1 < _NCH:
            copies.append(pltpu.async_copy(
                xs.at[sl, pl.ds(base + (j + 1) * _CW, _CW)],
                bufs[(j + 1) % 2], sems[(j + 1) % 2]))
        copies[j].wait()
        buf = bufs[j % 2]

        def grp(g, carry2, _j=j, _buf=buf):
            lvec = idx_v[_j * (_CH // 128) + g // 8, pl.ds((g % 8) * 16, 16)] * _DS
            for t in range(16):
                idx = lax.broadcast(lvec[t], (16,)) + cols0
                vec = _buf[pl.ds((g * 16 + t) * _DS, 16)]
                plsc.addupdate_scatter(acc, [idx], vec)
            return carry2

        lax.fori_loop(0, _CH // 16, grp, 0)

    pltpu.sync_copy(acc, out.at[sl, p])


def _assign_body(craw_ref, x_ref, out_ref, cn_ref):
    @pl.when(pl.program_id(0) == 0)
    def _():
        slices = [craw_ref[q, 0] + craw_ref[q, 1] for q in range(_NS)]
        c = jnp.concatenate(slices, axis=1)
        norm = jnp.sqrt(jnp.sum(c * c, axis=1, keepdims=True))
        cn_ref[...] = c / (norm + 1e-12)

    sims = jax.lax.dot_general(
        x_ref[...], cn_ref[...],
        dimension_numbers=(((1,), (1,)), ((), ())),
        preferred_element_type=jnp.float32)
    out_ref[...] = jnp.argmax(sims, axis=1).astype(jnp.int32).reshape(1, 1, _ROWS)


def _tc_assign(xn, craw):
    out = pl.pallas_call(
        _assign_body,
        grid=(_NBLK,),
        in_specs=[
            pl.BlockSpec((_NS, 2, _K, _DS), lambda i: (0, 0, 0, 0)),
            pl.BlockSpec((_ROWS, _D), lambda i: (i, 0)),
        ],
        out_specs=pl.BlockSpec((1, 1, _ROWS), lambda i: (i, 0, 0)),
        out_shape=jax.ShapeDtypeStruct((_NBLK, 1, _ROWS), jnp.int32),
        scratch_shapes=[pltpu.VMEM((_K, _D), jnp.float32)],
    )(craw, xn)
    return out.reshape(_N)


def _norm(v, axis=-1, eps=1e-12):
    return v / (jnp.linalg.norm(v, axis=axis, keepdims=True) + eps)


def kernel(embeddings, batch_indices):
    key = jax.random.key(42)
    embs = embeddings.reshape(-1, embeddings.shape[-1])
    n = embs.shape[0]
    key, sub = jax.random.split(key)
    init_labels = jax.random.randint(sub, (n,), 0, _K)
    x = _norm(embs)
    xs = x.reshape(_N, _NS, _DS).transpose(1, 0, 2).reshape(_NS, _N * _DS)
    zc = jnp.zeros((_AW,), jnp.float32)
    labels = init_labels
    for _ in range(_ITERS):
        lab2d = labels.reshape(_N // 128, 128)
        craw = _sc_segsum(xs, lab2d, zc)
        labels = _tc_assign(x, craw.reshape(_NS, 2, _K, _DS))
    return (labels, batch_indices.astype(labels.dtype))


# speedup vs baseline: 1.7204x; 1.2473x over previous
import functools

import jax
import jax.numpy as jnp
from jax import lax
from jax.experimental import pallas as pl
from jax.experimental.pallas import tpu as pltpu
from jax.experimental.pallas import tpu_sc as plsc

_K = 1024
_ITERS = 10
_N = 16384
_D = 256
_NS = 16
_DS = _D // _NS
_PPT = _N // 2
_CH = 2048
_NCH = _PPT // _CH
_CW = _CH * _DS
_AW = _K * _DS

_ROWS = 1024
_NBLK = _N // _ROWS


_sc_mesh = plsc.VectorSubcoreMesh(core_axis_name="c", subcore_axis_name="s")


@functools.partial(
    pl.kernel,
    out_type=jax.ShapeDtypeStruct((_NS, 2, _AW), jnp.float32),
    mesh=_sc_mesh,
    compiler_params=pltpu.CompilerParams(needs_layout_passes=False),
    scratch_types=[
        pltpu.VMEM((_CW,), jnp.float32),
        pltpu.VMEM((_CW,), jnp.float32),
        pltpu.VMEM((_PPT // 128, 128), jnp.int32),
        pltpu.VMEM((_AW,), jnp.float32),
        pltpu.SemaphoreType.DMA,
        pltpu.SemaphoreType.DMA,
    ],
)
def _sc_segsum(xs, lab2d, zc, out, data_a, data_b, idx_v, acc, sem0, sem1):
    c = lax.axis_index("c")
    s = lax.axis_index("s")
    w = c * 16 + s
    sl = w // 2
    p = w % 2
    base = p * _PPT * _DS
    sems = (sem0, sem1)
    bufs = (data_a, data_b)
    copies = [
        pltpu.async_copy(
            xs.at[sl, pl.ds(base + j * _CW, _CW)], bufs[j % 2], sems[j % 2])
        for j in range(1)
    ]
    pltpu.sync_copy(zc, acc)
    pltpu.sync_copy(lab2d.at[pl.ds(p * (_PPT // 128), _PPT // 128)], idx_v)
    cols0 = lax.iota(jnp.int32, 16)

    for j in range(_NCH):
        if j + 1 < _NCH:
            copies.append(pltpu.async_copy(
                xs.at[sl, pl.ds(base + (j + 1) * _CW, _CW)],
                bufs[(j + 1) % 2], sems[(j + 1) % 2]))
        copies[j].wait()
        buf = bufs[j % 2]

        def grp(g, _j=j, _buf=buf):
            lvec = idx_v[_j * (_CH // 128) + g // 8, pl.ds((g % 8) * 16, 16)] * _DS
            for t in range(16):
                idx = lax.broadcast(lvec[t], (16,)) + cols0
                vec = _buf[pl.ds((g * 16 + t) * _DS, 16)]
                plsc.addupdate_scatter(acc, [idx], vec)

        plsc.parallel_loop(0, _CH // 16, step=1, unroll=2)(grp)

    pltpu.sync_copy(acc, out.at[sl, p])


def _assign_body(craw_ref, x_ref, out_ref, cn_ref):
    @pl.when(pl.program_id(0) == 0)
    def _():
        slices = [craw_ref[q, 0] + craw_ref[q, 1] for q in range(_NS)]
        c = jnp.concatenate(slices, axis=1)
        norm = jnp.sqrt(jnp.sum(c * c, axis=1, keepdims=True))
        cn_ref[...] = c / (norm + 1e-12)

    sims = jax.lax.dot_general(
        x_ref[...], cn_ref[...],
        dimension_numbers=(((1,), (1,)), ((), ())),
        preferred_element_type=jnp.float32)
    out_ref[...] = jnp.argmax(sims, axis=1).astype(jnp.int32).reshape(1, 1, _ROWS)


def _tc_assign(xn, craw):
    out = pl.pallas_call(
        _assign_body,
        grid=(_NBLK,),
        in_specs=[
            pl.BlockSpec((_NS, 2, _K, _DS), lambda i: (0, 0, 0, 0)),
            pl.BlockSpec((_ROWS, _D), lambda i: (i, 0)),
        ],
        out_specs=pl.BlockSpec((1, 1, _ROWS), lambda i: (i, 0, 0)),
        out_shape=jax.ShapeDtypeStruct((_NBLK, 1, _ROWS), jnp.int32),
        scratch_shapes=[pltpu.VMEM((_K, _D), jnp.float32)],
    )(craw, xn)
    return out.reshape(_N)


def _norm(v, axis=-1, eps=1e-12):
    return v / (jnp.linalg.norm(v, axis=axis, keepdims=True) + eps)


def kernel(embeddings, batch_indices):
    key = jax.random.key(42)
    embs = embeddings.reshape(-1, embeddings.shape[-1])
    n = embs.shape[0]
    key, sub = jax.random.split(key)
    init_labels = jax.random.randint(sub, (n,), 0, _K)
    x = _norm(embs)
    xs = x.reshape(_N, _NS, _DS).transpose(1, 0, 2).reshape(_NS, _N * _DS)
    zc = jnp.zeros((_AW,), jnp.float32)
    labels = init_labels
    for _ in range(_ITERS):
        lab2d = labels.reshape(_N // 128, 128)
        craw = _sc_segsum(xs, lab2d, zc)
        labels = _tc_assign(x, craw.reshape(_NS, 2, _K, _DS))
    return (labels, batch_indices.astype(labels.dtype))
